# baseline (speedup 1.0000x reference)
import functools

import jax
import jax.numpy as jnp
from jax import lax
from jax.experimental import pallas as pl
from jax.experimental.pallas import tpu as pltpu
from jax.experimental.pallas import tpu_sc as plsc

D = 64
NC, NS = 2, 16
NW = NC * NS
CHUNK = 128


@functools.lru_cache(maxsize=None)
def _build(total):
    assert total % (NW * CHUNK) == 0
    n_per_w = total // NW
    n_chunks = n_per_w // CHUNK
    mesh = plsc.VectorSubcoreMesh(core_axis_name="c", subcore_axis_name="s")

    @functools.partial(
        pl.kernel,
        mesh=mesh,
        out_type=jax.ShapeDtypeStruct((total, D), jnp.float32),
        scratch_types=[
            pltpu.VMEM((n_chunks, CHUNK), jnp.int32),
            pltpu.VMEM((CHUNK, D), jnp.float32),
            pltpu.SemaphoreType.DMA,
        ],
        compiler_params=pltpu.CompilerParams(use_tc_tiling_on_sc=False),
    )
    def gather_kernel(idx_hbm, table_hbm, out_hbm, idx_v, rows_v, sem):
        wid = lax.axis_index("s") * NC + lax.axis_index("c")
        base = wid * n_per_w
        pltpu.sync_copy(idx_hbm.at[wid], idx_v)

        def body(j, carry):
            pltpu.async_copy(table_hbm.at[idx_v.at[j]], rows_v, sem).wait()
            pltpu.sync_copy(rows_v, out_hbm.at[pl.ds(base + j * CHUNK, CHUNK)])
            return carry

        lax.fori_loop(0, n_chunks, body, 0)

    return gather_kernel


def kernel(word_batch, table):
    b, s = word_batch.shape
    total = b * s
    idx = word_batch.astype(jnp.int32).reshape(NW, total // (NW * CHUNK), CHUNK)
    out = _build(total)(idx, table)
    return out.reshape(b, s, D)


# speedup vs baseline: 1.1170x; 1.1170x over previous
import functools

import jax
import jax.numpy as jnp
from jax import lax
from jax.experimental import pallas as pl
from jax.experimental.pallas import tpu as pltpu
from jax.experimental.pallas import tpu_sc as plsc

D = 64
NC, NS = 2, 16
NW = NC * NS
CHUNK = 128
NBUF = 8


@functools.lru_cache(maxsize=None)
def _build(total):
    assert total % (NW * CHUNK) == 0
    n_per_w = total // NW
    n_chunks = n_per_w // CHUNK
    assert n_chunks % NBUF == 0
    n_rings = n_chunks // NBUF
    mesh = plsc.VectorSubcoreMesh(core_axis_name="c", subcore_axis_name="s")

    @functools.partial(
        pl.kernel,
        mesh=mesh,
        out_type=jax.ShapeDtypeStruct((total, D), jnp.float32),
        scratch_types=[
            pltpu.VMEM((n_chunks, CHUNK), jnp.int32),
            pltpu.VMEM((NBUF, CHUNK, D), jnp.float32),
            pltpu.SemaphoreType.DMA((NBUF,)),
            pltpu.SemaphoreType.DMA((NBUF,)),
        ],
        compiler_params=pltpu.CompilerParams(use_tc_tiling_on_sc=False),
    )
    def gather_kernel(idx_hbm, table_hbm, out_hbm, idx_v, rows_v, gsem, ssem):
        wid = lax.axis_index("s") * NC + lax.axis_index("c")
        base = wid * n_per_w
        pltpu.sync_copy(idx_hbm.at[wid], idx_v)

        def g_copy(j, b):
            return pltpu.make_async_copy(
                table_hbm.at[idx_v.at[j]], rows_v.at[b], gsem.at[b])

        def s_copy(j, b):
            return pltpu.make_async_copy(
                rows_v.at[b], out_hbm.at[pl.ds(base + j * CHUNK, CHUNK)],
                ssem.at[b])

        Dg = NBUF // 2

        for v in range(Dg):
            g_copy(v, v % NBUF).start()
        for v in range(Dg, NBUF):
            g_copy(v - Dg, (v - Dg) % NBUF).wait()
            s_copy(v - Dg, (v - Dg) % NBUF).start()
            g_copy(v, v % NBUF).start()

        def ring(r, carry):
            for b in range(NBUF):
                v = r * NBUF + b
                bg = (b + NBUF - Dg) % NBUF
                g_copy(v - Dg, bg).wait()
                s_copy(v - Dg, bg).start()
                s_copy(v - NBUF, b).wait()
                g_copy(v, b).start()
            return carry

        lax.fori_loop(1, n_rings, ring, 0)

        for k in range(Dg):
            v = n_chunks + k
            bg = (v - Dg) % NBUF
            g_copy(v - Dg, bg).wait()
            s_copy(v - Dg, bg).start()
            s_copy(v - NBUF, v % NBUF).wait()
        for k in range(Dg):
            j = n_chunks - Dg + k
            s_copy(j, j % NBUF).wait()

    return gather_kernel


def kernel(word_batch, table):
    b, s = word_batch.shape
    total = b * s
    idx = word_batch.astype(jnp.int32).reshape(NW, total // (NW * CHUNK), CHUNK)
    out = _build(total)(idx, table)
    return out.reshape(b, s, D)
